# baseline (speedup 1.0000x reference)
import functools

import jax
import jax.numpy as jnp
from jax.experimental import pallas as pl
from jax.experimental.pallas import tpu as pltpu

D = 128
H = 256
THRESH = 0.99
SLOPE = 0.2
LOG2E = 1.4426950408889634


def _lk(x):
    return jnp.maximum(x, SLOPE * x)


def _dec_apply(r, wi, bi, wm, bm, wo, bo):
    z = jax.lax.dot(r.astype(jnp.bfloat16), wi.astype(jnp.bfloat16),
                    preferred_element_type=jnp.float32)
    z = jnp.maximum(z + bi, 0.0)
    z2 = jax.lax.dot(z.astype(jnp.bfloat16), wm.astype(jnp.bfloat16),
                     preferred_element_type=jnp.float32)
    z2 = jnp.maximum(z2 + bm, 0.0) + z
    o = jax.lax.dot(z2.astype(jnp.bfloat16), wo.astype(jnp.bfloat16),
                    preferred_element_type=jnp.float32)
    return o + bo


def _colsum(x):
    return jax.lax.dot(jnp.ones((1, x.shape[0]), jnp.float32), x,
                       preferred_element_type=jnp.float32)


def _row_body(gj, ti, tj, n, has_extra, has_res, leaky_out, has_dec,
              emit_mask, mask_input, *refs):
    x_ref, w_ref, aq_ref, ak_ref, a_ref = refs[:5]
    idx = 5
    e_ref = r_ref = None
    if has_extra:
        e_ref = refs[idx]
        idx += 1
    if has_res:
        r_ref = refs[idx]
        idx += 1
    dec = None
    if has_dec:
        dec = refs[idx:idx + 6]
        idx += 6
    o_ref = refs[idx]
    idx += 1
    m8_ref = None
    if emit_mask:
        m8_ref = refs[idx]
        idx += 1
    (h_ref, qa_ref, qb_ref, kc_ref, kd_ref, hsum_ref, l_ref, acc_ref) = \
        refs[idx:]
    i = pl.program_id(0)
    j = pl.program_id(1)
    rds = pl.ds(i * ti, ti)
    cds = pl.ds(j * tj, tj)

    @pl.when((i == 0) & (j == 0))
    def _():
        w = w_ref[...]
        x = x_ref[...]
        q = jax.lax.dot(
            x, jax.lax.dot(w, aq_ref[...], preferred_element_type=jnp.float32),
            preferred_element_type=jnp.float32)
        k = jax.lax.dot(
            x, jax.lax.dot(w, ak_ref[...], preferred_element_type=jnp.float32),
            preferred_element_type=jnp.float32)
        km = jnp.max(k, keepdims=True)
        li = _lk(q + km)
        qa_ref[...] = (q - li) * LOG2E
        qb_ref[...] = (SLOPE * q - li) * LOG2E
        kc_ref[...] = (k * LOG2E).T
        kd_ref[...] = (SLOPE * LOG2E * k).T
        hsum_ref[...] = jax.lax.dot(_colsum(x), w,
                                    preferred_element_type=jnp.float32)

    @pl.when(i == 0)
    def _():
        h_ref[cds, :] = jax.lax.dot(x_ref[cds, :], w_ref[...],
                                    preferred_element_type=jnp.float32)

    @pl.when(j == 0)
    def _():
        acc_ref[...] = jnp.zeros(acc_ref.shape, jnp.float32)
        l_ref[...] = jnp.zeros(l_ref.shape, jnp.float32)

    arg = jnp.maximum(qa_ref[rds, :] + kc_ref[:, cds],
                      qb_ref[rds, :] + kd_ref[:, cds])
    if mask_input:
        mask = a_ref[...] != 0
    else:
        mask = a_ref[...] > THRESH
    e = jnp.where(mask, jnp.exp2(arg), 0.0)
    if emit_mask:
        m8_ref[...] = mask.astype(jnp.int8)
    acc_ref[...] += jax.lax.dot(e, h_ref[cds, :],
                                preferred_element_type=jnp.float32)
    l_ref[...] += jax.lax.dot(e, jnp.ones((tj, 1), jnp.float32),
                              preferred_element_type=jnp.float32)

    @pl.when(j == gj - 1)
    def _():
        l = l_ref[...]
        r = jnp.where(l > 0, acc_ref[...] / (l + 1e-9),
                      hsum_ref[...] / (n + 1e-9))
        if has_extra:
            r = r + e_ref[...]
        if leaky_out:
            r = jnp.maximum(r, SLOPE * r)
        if has_res:
            r = r + r_ref[...]
        if has_dec:
            r = _dec_apply(r, dec[0][...], dec[1][...], dec[2][...],
                           dec[3][...], dec[4][...], dec[5][...])
        o_ref[...] = r


def _row_flash(x, w, aq, ak, adj, extra=None, residual=None, leaky_out=False,
               dec=None, emit_mask=False, mask_input=False):
    n = adj.shape[0]
    ti = min(256, n)
    tj = min(4096, n)
    gi, gj = n // ti, n // tj
    inputs = [x, w, aq.reshape(D, 1), ak.reshape(D, 1), adj]
    specs = [
        pl.BlockSpec((n, D), lambda i, j: (0, 0)),
        pl.BlockSpec((D, D), lambda i, j: (0, 0)),
        pl.BlockSpec((D, 1), lambda i, j: (0, 0)),
        pl.BlockSpec((D, 1), lambda i, j: (0, 0)),
        pl.BlockSpec((ti, tj), lambda i, j: (i, j)),
    ]
    if extra is not None:
        inputs.append(extra)
        specs.append(pl.BlockSpec((ti, D), lambda i, j: (i, 0)))
    if residual is not None:
        inputs.append(residual)
        specs.append(pl.BlockSpec((ti, D), lambda i, j: (i, 0)))
    if dec is not None:
        inputs.extend(dec)
        specs.extend([
            pl.BlockSpec((D, H), lambda i, j: (0, 0)),
            pl.BlockSpec((1, H), lambda i, j: (0, 0)),
            pl.BlockSpec((H, H), lambda i, j: (0, 0)),
            pl.BlockSpec((1, H), lambda i, j: (0, 0)),
            pl.BlockSpec((H, D), lambda i, j: (0, 0)),
            pl.BlockSpec((1, D), lambda i, j: (0, 0)),
        ])
    body = functools.partial(_row_body, gj, ti, tj, float(n),
                             extra is not None, residual is not None,
                             leaky_out, dec is not None, emit_mask,
                             mask_input)
    out_specs = [pl.BlockSpec((ti, D), lambda i, j: (i, 0))]
    out_shape = [jax.ShapeDtypeStruct((n, D), jnp.float32)]
    if emit_mask:
        out_specs.append(pl.BlockSpec((ti, tj), lambda i, j: (i, j)))
        out_shape.append(jax.ShapeDtypeStruct((n, n), jnp.int8))
    res = pl.pallas_call(
        body,
        grid=(gi, gj),
        in_specs=specs,
        out_specs=out_specs,
        out_shape=out_shape,
        scratch_shapes=[
            pltpu.VMEM((n, D), jnp.float32),
            pltpu.VMEM((n, 1), jnp.float32),
            pltpu.VMEM((n, 1), jnp.float32),
            pltpu.VMEM((1, n), jnp.float32),
            pltpu.VMEM((1, n), jnp.float32),
            pltpu.VMEM((1, D), jnp.float32),
            pltpu.VMEM((ti, 1), jnp.float32),
            pltpu.VMEM((ti, D), jnp.float32),
        ],
    )(*inputs)
    return res if emit_mask else res[0]


def _col_body(gs, ts, tt, ns,
              xs_ref, ws_ref, avs_ref, xt_ref, wt_ref, avt_ref, a_ref,
              o_ref, hsT_ref, qc_ref, qd_ref, ka_ref, kb_ref, hsumT_ref,
              acc_ref, l_ref):
    t = pl.program_id(0)
    s_id = pl.program_id(1)
    sds = pl.ds(s_id * ts, ts)
    tds = pl.ds(t * tt, tt)

    @pl.when((t == 0) & (s_id == 0))
    def _():
        ws = ws_ref[...]
        xs = xs_ref[...]
        q = jax.lax.dot(
            xs, jax.lax.dot(ws, avs_ref[...],
                            preferred_element_type=jnp.float32),
            preferred_element_type=jnp.float32)
        qm = jnp.max(q, keepdims=True)
        k = jax.lax.dot(
            xt_ref[...],
            jax.lax.dot(wt_ref[...], avt_ref[...],
                        preferred_element_type=jnp.float32),
            preferred_element_type=jnp.float32)
        lt = _lk(qm + k)
        qc_ref[...] = q * LOG2E
        qd_ref[...] = SLOPE * LOG2E * q
        ka_ref[...] = ((k - lt) * LOG2E).T
        kb_ref[...] = ((SLOPE * k - lt) * LOG2E).T
        hsumT_ref[...] = jax.lax.dot(_colsum(xs), ws,
                                     preferred_element_type=jnp.float32).T

    @pl.when(t == 0)
    def _():
        hsT_ref[:, sds] = jax.lax.dot(
            xs_ref[sds, :], ws_ref[...],
            preferred_element_type=jnp.float32).T

    @pl.when(s_id == 0)
    def _():
        acc_ref[...] = jnp.zeros(acc_ref.shape, jnp.float32)
        l_ref[...] = jnp.zeros(l_ref.shape, jnp.float32)

    arg = jnp.maximum(qc_ref[sds, :] + ka_ref[:, tds],
                      qd_ref[sds, :] + kb_ref[:, tds])
    e = jnp.where(a_ref[...] != 0, jnp.exp2(arg), 0.0)
    acc_ref[...] += jax.lax.dot(hsT_ref[:, sds], e,
                                preferred_element_type=jnp.float32)
    l_ref[...] += jax.lax.dot(jnp.ones((1, ts), jnp.float32), e,
                              preferred_element_type=jnp.float32)

    @pl.when(s_id == gs - 1)
    def _():
        l = l_ref[...]
        r = jnp.where(l > 0, acc_ref[...] / (l + 1e-9),
                      hsumT_ref[...] / (ns + 1e-9))
        o_ref[...] = r.T


def _col_flash(xs, ws, avs, xt, wt, avt, adj):
    ns, nt = adj.shape
    ts = min(4096, ns)
    tt = min(256, nt)
    gs, gt = ns // ts, nt // tt
    body = functools.partial(_col_body, gs, ts, tt, float(ns))
    return pl.pallas_call(
        body,
        grid=(gt, gs),
        in_specs=[
            pl.BlockSpec((ns, D), lambda t, s: (0, 0)),
            pl.BlockSpec((D, D), lambda t, s: (0, 0)),
            pl.BlockSpec((D, 1), lambda t, s: (0, 0)),
            pl.BlockSpec((nt, D), lambda t, s: (0, 0)),
            pl.BlockSpec((D, D), lambda t, s: (0, 0)),
            pl.BlockSpec((D, 1), lambda t, s: (0, 0)),
            pl.BlockSpec((ts, tt), lambda t, s: (s, t)),
        ],
        out_specs=pl.BlockSpec((tt, D), lambda t, s: (t, 0)),
        out_shape=jax.ShapeDtypeStruct((nt, D), jnp.float32),
        scratch_shapes=[
            pltpu.VMEM((D, ns), jnp.float32),
            pltpu.VMEM((ns, 1), jnp.float32),
            pltpu.VMEM((ns, 1), jnp.float32),
            pltpu.VMEM((1, nt), jnp.float32),
            pltpu.VMEM((1, nt), jnp.float32),
            pltpu.VMEM((D, 1), jnp.float32),
            pltpu.VMEM((D, tt), jnp.float32),
            pltpu.VMEM((1, tt), jnp.float32),
        ],
    )(xs, ws, avs.reshape(D, 1), xt, wt, avt.reshape(D, 1), adj)


def _dual_body(gi, gj, ti, tj, ns, nt, has_extra_s, leaky_s, leaky_t, *refs):
    xs_ref, ws_ref, avs_ref, xt_ref, wt_ref, avt_ref, a_ref = refs[:7]
    idx = 7
    es_ref = None
    if has_extra_s:
        es_ref = refs[idx]
        idx += 1
    os_ref, ot_ref, m8_ref = refs[idx], refs[idx + 1], refs[idx + 2]
    (qa_ref, qb_ref, kc_ref, kd_ref, w_ref, f_ref, hssT_ref,
     ht_ref, hts_ref, lr_ref, lc_ref, otT_ref, hsT_ref) = refs[idx + 3:]
    i = pl.program_id(0)
    j = pl.program_id(1)
    rds = pl.ds(i * ti, ti)
    cds = pl.ds(j * tj, tj)

    @pl.when((i == 0) & (j == 0))
    def _():
        ws = ws_ref[...]
        wt = wt_ref[...]
        xs = xs_ref[...]
        xt = xt_ref[...]
        q = jax.lax.dot(
            xs, jax.lax.dot(ws, avs_ref[...],
                            preferred_element_type=jnp.float32),
            preferred_element_type=jnp.float32)
        k = jax.lax.dot(
            xt, jax.lax.dot(wt, avt_ref[...],
                            preferred_element_type=jnp.float32),
            preferred_element_type=jnp.float32)
        qm = jnp.max(q, keepdims=True)
        km = jnp.max(k, keepdims=True)
        lmax = _lk(qm + km)
        li = _lk(q + km)
        lt = _lk(qm + k)
        qa_ref[...] = (q - li) * LOG2E
        qb_ref[...] = (SLOPE * q - li) * LOG2E
        kc_ref[...] = (k * LOG2E).T
        kd_ref[...] = (SLOPE * LOG2E * k).T
        w_ref[...] = jnp.exp(li - lmax).T
        f_ref[...] = jnp.exp(lmax - lt).T
        hssT_ref[...] = jax.lax.dot(_colsum(xs), ws,
                                    preferred_element_type=jnp.float32).T
        hts_ref[...] = jax.lax.dot(_colsum(xt), wt,
                                   preferred_element_type=jnp.float32)

    @pl.when(i == 0)
    def _():
        ht_ref[cds, :] = jax.lax.dot(xt_ref[cds, :], wt_ref[...],
                                     preferred_element_type=jnp.float32)
        otT_ref[:, cds] = jnp.zeros((D, tj), jnp.float32)
        lc_ref[:, cds] = jnp.zeros((1, tj), jnp.float32)

    @pl.when(j == 0)
    def _():
        os_ref[...] = jnp.zeros(os_ref.shape, jnp.float32)
        lr_ref[...] = jnp.zeros(lr_ref.shape, jnp.float32)
        hs_tile = jax.lax.dot(xs_ref[rds, :], ws_ref[...],
                              preferred_element_type=jnp.float32)
        hsT_ref[...] = hs_tile.T * w_ref[:, rds]

    arg = jnp.maximum(qa_ref[rds, :] + kc_ref[:, cds],
                      qb_ref[rds, :] + kd_ref[:, cds])
    mask = a_ref[...] > THRESH
    e = jnp.where(mask, jnp.exp2(arg), 0.0)
    m8_ref[...] = mask.astype(jnp.int8)

    os_ref[...] += jax.lax.dot(e, ht_ref[cds, :],
                               preferred_element_type=jnp.float32)
    lr_ref[...] += jax.lax.dot(e, jnp.ones((tj, 1), jnp.float32),
                               preferred_element_type=jnp.float32)

    @pl.when(j == gj - 1)
    def _():
        l = lr_ref[...]
        r = jnp.where(l > 0, os_ref[...] / (l + 1e-9),
                      hts_ref[...] / (nt + 1e-9))
        if has_extra_s:
            r = r + es_ref[...]
        if leaky_s:
            r = jnp.maximum(r, SLOPE * r)
        os_ref[...] = r

    otT_ref[:, cds] += jax.lax.dot(hsT_ref[...], e,
                                   preferred_element_type=jnp.float32)
    lc_ref[:, cds] += jax.lax.dot(w_ref[:, rds], e,
                                  preferred_element_type=jnp.float32)

    @pl.when(i == gi - 1)
    def _():
        f = f_ref[:, cds]
        lf = lc_ref[:, cds] * f
        rt = jnp.where(lf > 0, (otT_ref[:, cds] * f) / (lf + 1e-9),
                       hssT_ref[...] / (ns + 1e-9))
        if leaky_t:
            rt = jnp.maximum(rt, SLOPE * rt)
        ot_ref[cds, :] = rt.T


def _dual_flash(xs, ws, avs, xt, wt, avt, adj,
                extra_s=None, leaky_s=False, leaky_t=False):
    ns, nt = adj.shape
    ti = min(256, ns)
    tj = min(4096, nt)
    gi, gj = ns // ti, nt // tj
    inputs = [xs, ws, avs.reshape(D, 1), xt, wt, avt.reshape(D, 1), adj]
    specs = [
        pl.BlockSpec((ns, D), lambda i, j: (0, 0)),
        pl.BlockSpec((D, D), lambda i, j: (0, 0)),
        pl.BlockSpec((D, 1), lambda i, j: (0, 0)),
        pl.BlockSpec((nt, D), lambda i, j: (0, 0)),
        pl.BlockSpec((D, D), lambda i, j: (0, 0)),
        pl.BlockSpec((D, 1), lambda i, j: (0, 0)),
        pl.BlockSpec((ti, tj), lambda i, j: (i, j)),
    ]
    if extra_s is not None:
        inputs.append(extra_s)
        specs.append(pl.BlockSpec((ti, D), lambda i, j: (i, 0)))
    body = functools.partial(_dual_body, gi, gj, ti, tj, float(ns), float(nt),
                             extra_s is not None, leaky_s, leaky_t)
    return pl.pallas_call(
        body,
        grid=(gi, gj),
        in_specs=specs,
        out_specs=[
            pl.BlockSpec((ti, D), lambda i, j: (i, 0)),
            pl.BlockSpec((nt, D), lambda i, j: (0, 0)),
            pl.BlockSpec((ti, tj), lambda i, j: (i, j)),
        ],
        out_shape=[
            jax.ShapeDtypeStruct((ns, D), jnp.float32),
            jax.ShapeDtypeStruct((nt, D), jnp.float32),
            jax.ShapeDtypeStruct((ns, nt), jnp.int8),
        ],
        scratch_shapes=[
            pltpu.VMEM((ns, 1), jnp.float32),
            pltpu.VMEM((ns, 1), jnp.float32),
            pltpu.VMEM((1, nt), jnp.float32),
            pltpu.VMEM((1, nt), jnp.float32),
            pltpu.VMEM((1, ns), jnp.float32),
            pltpu.VMEM((1, nt), jnp.float32),
            pltpu.VMEM((D, 1), jnp.float32),
            pltpu.VMEM((nt, D), jnp.float32),
            pltpu.VMEM((1, D), jnp.float32),
            pltpu.VMEM((ti, 1), jnp.float32),
            pltpu.VMEM((1, nt), jnp.float32),
            pltpu.VMEM((D, nt), jnp.float32),
            pltpu.VMEM((D, ti), jnp.float32),
        ],
    )(*inputs)


def kernel(x_0, x_1, x_2, a0, a1, coa2, b1, b2, params):
    p = params

    def dec_params(pre):
        return (p[pre + 'W_in'], p[pre + 'b_in'].reshape(1, H),
                p[pre + 'W_mid'], p[pre + 'b_mid'].reshape(1, H),
                p[pre + 'W_out'], p[pre + 'b_out'].reshape(1, D))

    s0, t1, m_b1 = _dual_flash(x_0, p['Ws_b1_1'], p['as_b1_1'],
                               x_1, p['Wt_b1_1'], p['at_b1_1'], b1)
    x1_l1, x2_l1, m_b2 = _dual_flash(x_1, p['Ws_b2_1'], p['as_b2_1'],
                                     x_2, p['Wt_b2_1'], p['at_b2_1'], b2,
                                     extra_s=t1, leaky_s=True, leaky_t=True)
    x0_l1, m_a0 = _row_flash(x_0, p['W_a0_1'], p['aq_a0_1'], p['ak_a0_1'],
                             a0, extra=s0, leaky_out=True, emit_mask=True)

    t1_2 = _col_flash(x0_l1, p['Ws_b1_2'], p['as_b1_2'],
                      x1_l1, p['Wt_b1_2'], p['at_b1_2'], m_b1)
    t2_2 = _col_flash(x1_l1, p['Ws_b2_2'], p['as_b2_2'],
                      x2_l1, p['Wt_b2_2'], p['at_b2_2'], m_b2)

    out0 = _row_flash(x0_l1, p['W_a0_2'], p['aq_a0_2'], p['ak_a0_2'], m_a0,
                      residual=x_0, leaky_out=True, dec=dec_params('d0_'),
                      mask_input=True)
    out1 = _row_flash(x1_l1, p['W_a1_2'], p['aq_a1_2'], p['ak_a1_2'], a1,
                      extra=t1_2, residual=x_1, leaky_out=True,
                      dec=dec_params('d1_'))
    out2 = _row_flash(x2_l1, p['W_coa2_2'], p['aq_coa2_2'], p['ak_coa2_2'],
                      coa2, extra=t2_2, residual=x_2, leaky_out=True,
                      dec=dec_params('d2_'))
    return (out0, out1, out2)


# speedup vs baseline: 1.0808x; 1.0808x over previous
import functools

import jax
import jax.numpy as jnp
from jax.experimental import pallas as pl
from jax.experimental.pallas import tpu as pltpu

D = 128
H = 256
THRESH = 0.99
SLOPE = 0.2
LOG2E = 1.4426950408889634


def _lk(x):
    return jnp.maximum(x, SLOPE * x)


def _dec_apply(r, wi, bi, wm, bm, wo, bo):
    z = jax.lax.dot(r.astype(jnp.bfloat16), wi.astype(jnp.bfloat16),
                    preferred_element_type=jnp.float32)
    z = jnp.maximum(z + bi, 0.0)
    z2 = jax.lax.dot(z.astype(jnp.bfloat16), wm.astype(jnp.bfloat16),
                     preferred_element_type=jnp.float32)
    z2 = jnp.maximum(z2 + bm, 0.0) + z
    o = jax.lax.dot(z2.astype(jnp.bfloat16), wo.astype(jnp.bfloat16),
                    preferred_element_type=jnp.float32)
    return o + bo


def _colsum(x):
    return jax.lax.dot(jnp.ones((1, x.shape[0]), jnp.float32), x,
                       preferred_element_type=jnp.float32)


def _row_body(gj, ti, tj, n, has_extra, has_res, leaky_out, has_dec,
              emit_mask, mask_input, *refs):
    x_ref, w_ref, aq_ref, ak_ref, a_ref = refs[:5]
    idx = 5
    e_ref = r_ref = None
    if has_extra:
        e_ref = refs[idx]
        idx += 1
    if has_res:
        r_ref = refs[idx]
        idx += 1
    dec = None
    if has_dec:
        dec = refs[idx:idx + 6]
        idx += 6
    o_ref = refs[idx]
    idx += 1
    m8_ref = None
    if emit_mask:
        m8_ref = refs[idx]
        idx += 1
    (h_ref, qa_ref, qb_ref, kc_ref, kd_ref, hsum_ref, l_ref, acc_ref) = \
        refs[idx:]
    i = pl.program_id(0)
    j = pl.program_id(1)
    rds = pl.ds(i * ti, ti)
    cds = pl.ds(j * tj, tj)

    @pl.when((i == 0) & (j == 0))
    def _():
        w = w_ref[...]
        x = x_ref[...]
        q = jax.lax.dot(
            x, jax.lax.dot(w, aq_ref[...], preferred_element_type=jnp.float32),
            preferred_element_type=jnp.float32)
        k = jax.lax.dot(
            x, jax.lax.dot(w, ak_ref[...], preferred_element_type=jnp.float32),
            preferred_element_type=jnp.float32)
        km = jnp.max(k, keepdims=True)
        li = _lk(q + km)
        qa_ref[...] = (q - li) * LOG2E
        qb_ref[...] = (SLOPE * q - li) * LOG2E
        kc_ref[...] = (k * LOG2E).T
        kd_ref[...] = (SLOPE * LOG2E * k).T
        hsum_ref[...] = jax.lax.dot(_colsum(x), w,
                                    preferred_element_type=jnp.float32)

    @pl.when(i == 0)
    def _():
        h_ref[cds, :] = jax.lax.dot(
            x_ref[cds, :], w_ref[...],
            preferred_element_type=jnp.float32).astype(jnp.bfloat16)

    @pl.when(j == 0)
    def _():
        acc_ref[...] = jnp.zeros(acc_ref.shape, jnp.float32)
        l_ref[...] = jnp.zeros(l_ref.shape, jnp.float32)

    arg = jnp.maximum(qa_ref[rds, :] + kc_ref[:, cds],
                      qb_ref[rds, :] + kd_ref[:, cds])
    if mask_input:
        mask = a_ref[...] != 0
    else:
        mask = a_ref[...] > THRESH
    e = jnp.where(mask, jnp.exp2(arg), 0.0).astype(jnp.bfloat16)
    if emit_mask:
        m8_ref[...] = mask.astype(jnp.int8)
    acc_ref[...] += jax.lax.dot(e, h_ref[cds, :],
                                preferred_element_type=jnp.float32)
    l_ref[...] += jax.lax.dot(e, jnp.ones((tj, 1), jnp.bfloat16),
                              preferred_element_type=jnp.float32)

    @pl.when(j == gj - 1)
    def _():
        l = l_ref[...]
        r = jnp.where(l > 0, acc_ref[...] / (l + 1e-9),
                      hsum_ref[...] / (n + 1e-9))
        if has_extra:
            r = r + e_ref[...]
        if leaky_out:
            r = jnp.maximum(r, SLOPE * r)
        if has_res:
            r = r + r_ref[...]
        if has_dec:
            r = _dec_apply(r, dec[0][...], dec[1][...], dec[2][...],
                           dec[3][...], dec[4][...], dec[5][...])
        o_ref[...] = r


def _row_flash(x, w, aq, ak, adj, extra=None, residual=None, leaky_out=False,
               dec=None, emit_mask=False, mask_input=False):
    n = adj.shape[0]
    ti = min(512, n)
    tj = min(4096, n)
    gi, gj = n // ti, n // tj
    inputs = [x, w, aq.reshape(D, 1), ak.reshape(D, 1), adj]
    specs = [
        pl.BlockSpec((n, D), lambda i, j: (0, 0)),
        pl.BlockSpec((D, D), lambda i, j: (0, 0)),
        pl.BlockSpec((D, 1), lambda i, j: (0, 0)),
        pl.BlockSpec((D, 1), lambda i, j: (0, 0)),
        pl.BlockSpec((ti, tj), lambda i, j: (i, j)),
    ]
    if extra is not None:
        inputs.append(extra)
        specs.append(pl.BlockSpec((ti, D), lambda i, j: (i, 0)))
    if residual is not None:
        inputs.append(residual)
        specs.append(pl.BlockSpec((ti, D), lambda i, j: (i, 0)))
    if dec is not None:
        inputs.extend(dec)
        specs.extend([
            pl.BlockSpec((D, H), lambda i, j: (0, 0)),
            pl.BlockSpec((1, H), lambda i, j: (0, 0)),
            pl.BlockSpec((H, H), lambda i, j: (0, 0)),
            pl.BlockSpec((1, H), lambda i, j: (0, 0)),
            pl.BlockSpec((H, D), lambda i, j: (0, 0)),
            pl.BlockSpec((1, D), lambda i, j: (0, 0)),
        ])
    body = functools.partial(_row_body, gj, ti, tj, float(n),
                             extra is not None, residual is not None,
                             leaky_out, dec is not None, emit_mask,
                             mask_input)
    out_specs = [pl.BlockSpec((ti, D), lambda i, j: (i, 0))]
    out_shape = [jax.ShapeDtypeStruct((n, D), jnp.float32)]
    if emit_mask:
        out_specs.append(pl.BlockSpec((ti, tj), lambda i, j: (i, j)))
        out_shape.append(jax.ShapeDtypeStruct((n, n), jnp.int8))
    res = pl.pallas_call(
        body,
        grid=(gi, gj),
        in_specs=specs,
        out_specs=out_specs,
        out_shape=out_shape,
        scratch_shapes=[
            pltpu.VMEM((n, D), jnp.bfloat16),
            pltpu.VMEM((n, 1), jnp.float32),
            pltpu.VMEM((n, 1), jnp.float32),
            pltpu.VMEM((1, n), jnp.float32),
            pltpu.VMEM((1, n), jnp.float32),
            pltpu.VMEM((1, D), jnp.float32),
            pltpu.VMEM((ti, 1), jnp.float32),
            pltpu.VMEM((ti, D), jnp.float32),
        ],
    )(*inputs)
    return res if emit_mask else res[0]


def _col_body(gs, ts, tt, ns,
              xs_ref, ws_ref, avs_ref, xt_ref, wt_ref, avt_ref, a_ref,
              o_ref, hsT_ref, qc_ref, qd_ref, ka_ref, kb_ref, hsumT_ref,
              acc_ref, l_ref):
    t = pl.program_id(0)
    s_id = pl.program_id(1)
    sds = pl.ds(s_id * ts, ts)
    tds = pl.ds(t * tt, tt)

    @pl.when((t == 0) & (s_id == 0))
    def _():
        ws = ws_ref[...]
        xs = xs_ref[...]
        q = jax.lax.dot(
            xs, jax.lax.dot(ws, avs_ref[...],
                            preferred_element_type=jnp.float32),
            preferred_element_type=jnp.float32)
        qm = jnp.max(q, keepdims=True)
        k = jax.lax.dot(
            xt_ref[...],
            jax.lax.dot(wt_ref[...], avt_ref[...],
                        preferred_element_type=jnp.float32),
            preferred_element_type=jnp.float32)
        lt = _lk(qm + k)
        qc_ref[...] = q * LOG2E
        qd_ref[...] = SLOPE * LOG2E * q
        ka_ref[...] = ((k - lt) * LOG2E).T
        kb_ref[...] = ((SLOPE * k - lt) * LOG2E).T
        hsumT_ref[...] = jax.lax.dot(_colsum(xs), ws,
                                     preferred_element_type=jnp.float32).T

    @pl.when(t == 0)
    def _():
        hsT_ref[:, sds] = jax.lax.dot(
            xs_ref[sds, :], ws_ref[...],
            preferred_element_type=jnp.float32).T.astype(jnp.bfloat16)

    @pl.when(s_id == 0)
    def _():
        acc_ref[...] = jnp.zeros(acc_ref.shape, jnp.float32)
        l_ref[...] = jnp.zeros(l_ref.shape, jnp.float32)

    arg = jnp.maximum(qc_ref[sds, :] + ka_ref[:, tds],
                      qd_ref[sds, :] + kb_ref[:, tds])
    e = jnp.where(a_ref[...] != 0, jnp.exp2(arg), 0.0).astype(jnp.bfloat16)
    acc_ref[...] += jax.lax.dot(hsT_ref[:, sds], e,
                                preferred_element_type=jnp.float32)
    l_ref[...] += jax.lax.dot(jnp.ones((1, ts), jnp.bfloat16), e,
                              preferred_element_type=jnp.float32)

    @pl.when(s_id == gs - 1)
    def _():
        l = l_ref[...]
        r = jnp.where(l > 0, acc_ref[...] / (l + 1e-9),
                      hsumT_ref[...] / (ns + 1e-9))
        o_ref[...] = r.T


def _col_flash(xs, ws, avs, xt, wt, avt, adj):
    ns, nt = adj.shape
    ts = min(4096, ns)
    tt = min(512, nt)
    gs, gt = ns // ts, nt // tt
    body = functools.partial(_col_body, gs, ts, tt, float(ns))
    return pl.pallas_call(
        body,
        grid=(gt, gs),
        in_specs=[
            pl.BlockSpec((ns, D), lambda t, s: (0, 0)),
            pl.BlockSpec((D, D), lambda t, s: (0, 0)),
            pl.BlockSpec((D, 1), lambda t, s: (0, 0)),
            pl.BlockSpec((nt, D), lambda t, s: (0, 0)),
            pl.BlockSpec((D, D), lambda t, s: (0, 0)),
            pl.BlockSpec((D, 1), lambda t, s: (0, 0)),
            pl.BlockSpec((ts, tt), lambda t, s: (s, t)),
        ],
        out_specs=pl.BlockSpec((tt, D), lambda t, s: (t, 0)),
        out_shape=jax.ShapeDtypeStruct((nt, D), jnp.float32),
        scratch_shapes=[
            pltpu.VMEM((D, ns), jnp.bfloat16),
            pltpu.VMEM((ns, 1), jnp.float32),
            pltpu.VMEM((ns, 1), jnp.float32),
            pltpu.VMEM((1, nt), jnp.float32),
            pltpu.VMEM((1, nt), jnp.float32),
            pltpu.VMEM((D, 1), jnp.float32),
            pltpu.VMEM((D, tt), jnp.float32),
            pltpu.VMEM((1, tt), jnp.float32),
        ],
    )(xs, ws, avs.reshape(D, 1), xt, wt, avt.reshape(D, 1), adj)


def _dual_body(gi, gj, ti, tj, ns, nt, has_extra_s, leaky_s, leaky_t, *refs):
    xs_ref, ws_ref, avs_ref, xt_ref, wt_ref, avt_ref, a_ref = refs[:7]
    idx = 7
    es_ref = None
    if has_extra_s:
        es_ref = refs[idx]
        idx += 1
    os_ref, ot_ref, m8_ref = refs[idx], refs[idx + 1], refs[idx + 2]
    (qa_ref, qb_ref, kc_ref, kd_ref, w_ref, f_ref, hssT_ref,
     ht_ref, hts_ref, lr_ref, lc_ref, otT_ref, hsT_ref) = refs[idx + 3:]
    i = pl.program_id(0)
    j = pl.program_id(1)
    rds = pl.ds(i * ti, ti)
    cds = pl.ds(j * tj, tj)

    @pl.when((i == 0) & (j == 0))
    def _():
        ws = ws_ref[...]
        wt = wt_ref[...]
        xs = xs_ref[...]
        xt = xt_ref[...]
        q = jax.lax.dot(
            xs, jax.lax.dot(ws, avs_ref[...],
                            preferred_element_type=jnp.float32),
            preferred_element_type=jnp.float32)
        k = jax.lax.dot(
            xt, jax.lax.dot(wt, avt_ref[...],
                            preferred_element_type=jnp.float32),
            preferred_element_type=jnp.float32)
        qm = jnp.max(q, keepdims=True)
        km = jnp.max(k, keepdims=True)
        lmax = _lk(qm + km)
        li = _lk(q + km)
        lt = _lk(qm + k)
        qa_ref[...] = (q - li) * LOG2E
        qb_ref[...] = (SLOPE * q - li) * LOG2E
        kc_ref[...] = (k * LOG2E).T
        kd_ref[...] = (SLOPE * LOG2E * k).T
        w_ref[...] = jnp.exp(li - lmax).T
        f_ref[...] = jnp.exp(lmax - lt).T
        hssT_ref[...] = jax.lax.dot(_colsum(xs), ws,
                                    preferred_element_type=jnp.float32).T
        hts_ref[...] = jax.lax.dot(_colsum(xt), wt,
                                   preferred_element_type=jnp.float32)

    @pl.when(i == 0)
    def _():
        ht_ref[cds, :] = jax.lax.dot(
            xt_ref[cds, :], wt_ref[...],
            preferred_element_type=jnp.float32).astype(jnp.bfloat16)
        otT_ref[:, cds] = jnp.zeros((D, tj), jnp.float32)
        lc_ref[:, cds] = jnp.zeros((1, tj), jnp.float32)

    @pl.when(j == 0)
    def _():
        os_ref[...] = jnp.zeros(os_ref.shape, jnp.float32)
        lr_ref[...] = jnp.zeros(lr_ref.shape, jnp.float32)
        hs_tile = jax.lax.dot(xs_ref[rds, :], ws_ref[...],
                              preferred_element_type=jnp.float32)
        hsT_ref[...] = (hs_tile.T * w_ref[:, rds]).astype(jnp.bfloat16)

    arg = jnp.maximum(qa_ref[rds, :] + kc_ref[:, cds],
                      qb_ref[rds, :] + kd_ref[:, cds])
    mask = a_ref[...] > THRESH
    e = jnp.where(mask, jnp.exp2(arg), 0.0).astype(jnp.bfloat16)
    m8_ref[...] = mask.astype(jnp.int8)

    os_ref[...] += jax.lax.dot(e, ht_ref[cds, :],
                               preferred_element_type=jnp.float32)
    lr_ref[...] += jax.lax.dot(e, jnp.ones((tj, 1), jnp.bfloat16),
                               preferred_element_type=jnp.float32)

    @pl.when(j == gj - 1)
    def _():
        l = lr_ref[...]
        r = jnp.where(l > 0, os_ref[...] / (l + 1e-9),
                      hts_ref[...] / (nt + 1e-9))
        if has_extra_s:
            r = r + es_ref[...]
        if leaky_s:
            r = jnp.maximum(r, SLOPE * r)
        os_ref[...] = r

    otT_ref[:, cds] += jax.lax.dot(hsT_ref[...], e,
                                   preferred_element_type=jnp.float32)
    lc_ref[:, cds] += jax.lax.dot(w_ref[:, rds].astype(jnp.bfloat16), e,
                                  preferred_element_type=jnp.float32)

    @pl.when(i == gi - 1)
    def _():
        f = f_ref[:, cds]
        lf = lc_ref[:, cds] * f
        rt = jnp.where(lf > 0, (otT_ref[:, cds] * f) / (lf + 1e-9),
                       hssT_ref[...] / (ns + 1e-9))
        if leaky_t:
            rt = jnp.maximum(rt, SLOPE * rt)
        ot_ref[cds, :] = rt.T


def _dual_flash(xs, ws, avs, xt, wt, avt, adj,
                extra_s=None, leaky_s=False, leaky_t=False):
    ns, nt = adj.shape
    ti = min(512, ns)
    tj = min(4096, nt)
    gi, gj = ns // ti, nt // tj
    inputs = [xs, ws, avs.reshape(D, 1), xt, wt, avt.reshape(D, 1), adj]
    specs = [
        pl.BlockSpec((ns, D), lambda i, j: (0, 0)),
        pl.BlockSpec((D, D), lambda i, j: (0, 0)),
        pl.BlockSpec((D, 1), lambda i, j: (0, 0)),
        pl.BlockSpec((nt, D), lambda i, j: (0, 0)),
        pl.BlockSpec((D, D), lambda i, j: (0, 0)),
        pl.BlockSpec((D, 1), lambda i, j: (0, 0)),
        pl.BlockSpec((ti, tj), lambda i, j: (i, j)),
    ]
    if extra_s is not None:
        inputs.append(extra_s)
        specs.append(pl.BlockSpec((ti, D), lambda i, j: (i, 0)))
    body = functools.partial(_dual_body, gi, gj, ti, tj, float(ns), float(nt),
                             extra_s is not None, leaky_s, leaky_t)
    return pl.pallas_call(
        body,
        grid=(gi, gj),
        in_specs=specs,
        out_specs=[
            pl.BlockSpec((ti, D), lambda i, j: (i, 0)),
            pl.BlockSpec((nt, D), lambda i, j: (0, 0)),
            pl.BlockSpec((ti, tj), lambda i, j: (i, j)),
        ],
        out_shape=[
            jax.ShapeDtypeStruct((ns, D), jnp.float32),
            jax.ShapeDtypeStruct((nt, D), jnp.float32),
            jax.ShapeDtypeStruct((ns, nt), jnp.int8),
        ],
        scratch_shapes=[
            pltpu.VMEM((ns, 1), jnp.float32),
            pltpu.VMEM((ns, 1), jnp.float32),
            pltpu.VMEM((1, nt), jnp.float32),
            pltpu.VMEM((1, nt), jnp.float32),
            pltpu.VMEM((1, ns), jnp.float32),
            pltpu.VMEM((1, nt), jnp.float32),
            pltpu.VMEM((D, 1), jnp.float32),
            pltpu.VMEM((nt, D), jnp.bfloat16),
            pltpu.VMEM((1, D), jnp.float32),
            pltpu.VMEM((ti, 1), jnp.float32),
            pltpu.VMEM((1, nt), jnp.float32),
            pltpu.VMEM((D, nt), jnp.float32),
            pltpu.VMEM((D, ti), jnp.bfloat16),
        ],
    )(*inputs)


def kernel(x_0, x_1, x_2, a0, a1, coa2, b1, b2, params):
    p = params

    def dec_params(pre):
        return (p[pre + 'W_in'], p[pre + 'b_in'].reshape(1, H),
                p[pre + 'W_mid'], p[pre + 'b_mid'].reshape(1, H),
                p[pre + 'W_out'], p[pre + 'b_out'].reshape(1, D))

    s0, t1, m_b1 = _dual_flash(x_0, p['Ws_b1_1'], p['as_b1_1'],
                               x_1, p['Wt_b1_1'], p['at_b1_1'], b1)
    x1_l1, x2_l1, m_b2 = _dual_flash(x_1, p['Ws_b2_1'], p['as_b2_1'],
                                     x_2, p['Wt_b2_1'], p['at_b2_1'], b2,
                                     extra_s=t1, leaky_s=True, leaky_t=True)
    x0_l1, m_a0 = _row_flash(x_0, p['W_a0_1'], p['aq_a0_1'], p['ak_a0_1'],
                             a0, extra=s0, leaky_out=True, emit_mask=True)

    t1_2 = _col_flash(x0_l1, p['Ws_b1_2'], p['as_b1_2'],
                      x1_l1, p['Wt_b1_2'], p['at_b1_2'], m_b1)
    t2_2 = _col_flash(x1_l1, p['Ws_b2_2'], p['as_b2_2'],
                      x2_l1, p['Wt_b2_2'], p['at_b2_2'], m_b2)

    out0 = _row_flash(x0_l1, p['W_a0_2'], p['aq_a0_2'], p['ak_a0_2'], m_a0,
                      residual=x_0, leaky_out=True, dec=dec_params('d0_'),
                      mask_input=True)
    out1 = _row_flash(x1_l1, p['W_a1_2'], p['aq_a1_2'], p['ak_a1_2'], a1,
                      extra=t1_2, residual=x_1, leaky_out=True,
                      dec=dec_params('d1_'))
    out2 = _row_flash(x2_l1, p['W_coa2_2'], p['aq_coa2_2'], p['ak_coa2_2'],
                      coa2, extra=t2_2, residual=x_2, leaky_out=True,
                      dec=dec_params('d2_'))
    return (out0, out1, out2)
